# baseline (speedup 1.0000x reference)
import functools

import jax
import jax.numpy as jnp
from jax import lax
from jax.experimental import pallas as pl
from jax.experimental.pallas import tpu as pltpu
from jax.experimental.pallas import tpu_sc as plsc

N = 4096
IN_DIM = 1024
HID = 2048
NCLS = 1024
E = 8
RH = 256

TM = 256
CAP = 2 * N + E * TM
NT = CAP // TM
NWORK = 32
NEG_INF = float("-inf")


@functools.cache
def _sc_mesh():
    return plsc.VectorSubcoreMesh(
        core_axis_name="c", subcore_axis_name="s", num_cores=2,
        num_subcores=16)


_BT = 1024


def _router_body(l_ref, i1_ref, i2_ref, g1_ref, g2_ref):
    logits = l_ref[...]
    ii = lax.broadcasted_iota(jnp.int32, (_BT, E), 1)
    m1 = jnp.max(logits, axis=1, keepdims=True)
    i1 = jnp.min(jnp.where(logits == m1, ii, E), axis=1, keepdims=True)
    l2 = jnp.where(ii == i1, NEG_INF, logits)
    m2 = jnp.max(l2, axis=1, keepdims=True)
    i2 = jnp.min(jnp.where(l2 == m2, ii, E), axis=1, keepdims=True)
    g1 = 1.0 / (1.0 + jnp.exp(m2 - m1))
    i1_ref[...] = i1
    i2_ref[...] = i2
    g1_ref[...] = g1
    g2_ref[...] = 1.0 - g1


def _run_router(x, w1, b1, w2, b2, w3, b3):
    h = jax.nn.relu(x @ w1.T + b1)
    h = jax.nn.relu(h @ w2.T + b2)
    logits = h @ w3.T + b3
    return pl.pallas_call(
        _router_body,
        grid=(N // _BT,),
        in_specs=[pl.BlockSpec((_BT, E), lambda i: (i, 0))],
        out_specs=[pl.BlockSpec((_BT, 1), lambda i: (i, 0))] * 4,
        out_shape=[
            jax.ShapeDtypeStruct((N, 1), jnp.int32),
            jax.ShapeDtypeStruct((N, 1), jnp.int32),
            jax.ShapeDtypeStruct((N, 1), jnp.float32),
            jax.ShapeDtypeStruct((N, 1), jnp.float32),
        ],
    )(logits)


_ROWS_PER_W = CAP // NWORK
_GCH = 40


@functools.cache
def _gather_rows_kernel():
    @functools.partial(
        pl.kernel,
        out_type=jax.ShapeDtypeStruct((CAP, IN_DIM), jnp.float32),
        mesh=_sc_mesh(),
        scratch_types=[
            pltpu.VMEM((2, _GCH), jnp.int32),
            pltpu.VMEM((2, _GCH, IN_DIM), jnp.float32),
            pltpu.SemaphoreType.DMA,
            pltpu.SemaphoreType.DMA,
        ],
    )
    def _gather_rows(x_hbm, idx_hbm, out_hbm, idx_v, rows_v, gsem, wsem):
        wid = lax.axis_index("s") * 2 + lax.axis_index("c")
        base = wid * _ROWS_PER_W
        nch = _ROWS_PER_W // _GCH
        pltpu.sync_copy(idx_hbm.at[pl.ds(base, _GCH)], idx_v.at[0])
        gd = pltpu.async_copy(x_hbm.at[idx_v.at[0]], rows_v.at[0], gsem)
        wd = [None, None]
        for c in range(nch):
            b = c & 1
            nb = 1 - b
            gd.wait()
            if c + 1 < nch:
                if wd[nb] is not None:
                    wd[nb].wait()
                pltpu.sync_copy(
                    idx_hbm.at[pl.ds(base + (c + 1) * _GCH, _GCH)],
                    idx_v.at[nb])
                gd = pltpu.async_copy(x_hbm.at[idx_v.at[nb]], rows_v.at[nb],
                                      gsem)
            wd[b] = pltpu.async_copy(
                rows_v.at[b], out_hbm.at[pl.ds(base + c * _GCH, _GCH)], wsem)
        for d in wd:
            if d is not None:
                d.wait()

    return _gather_rows


def _gemm_body(eid_ref, nv_ref, xs_ref, w1_ref, g_ref, b_ref, w2_ref,
               gate_ref, out_ref):
    i = pl.program_id(0)

    @pl.when(i < nv_ref[0])
    def _():
        xb = xs_ref[...].astype(jnp.bfloat16)
        h = lax.dot_general(xb, w1_ref[0], (((1,), (1,)), ((), ())),
                            preferred_element_type=jnp.float32)
        mu = jnp.mean(h, axis=1, keepdims=True)
        var = jnp.mean((h - mu) * (h - mu), axis=1, keepdims=True)
        h = (h - mu) * lax.rsqrt(var + 1e-5)
        h = h * g_ref[0] + b_ref[0]
        h = jnp.maximum(h, 0.0).astype(jnp.bfloat16)
        out = lax.dot_general(h, w2_ref[0], (((1,), (1,)), ((), ())),
                              preferred_element_type=jnp.float32)
        out_ref[...] = out * gate_ref[...]


def _run_experts(xs, expert_w1, ln_gamma, ln_beta, expert_w2, gate_s,
                 tile_eid, nvalid):
    grid_spec = pltpu.PrefetchScalarGridSpec(
        num_scalar_prefetch=2,
        grid=(NT,),
        in_specs=[
            pl.BlockSpec((TM, IN_DIM), lambda i, eid, nv: (i, 0)),
            pl.BlockSpec((1, HID, IN_DIM), lambda i, eid, nv: (eid[i], 0, 0)),
            pl.BlockSpec((1, 1, HID), lambda i, eid, nv: (eid[i], 0, 0)),
            pl.BlockSpec((1, 1, HID), lambda i, eid, nv: (eid[i], 0, 0)),
            pl.BlockSpec((1, NCLS, HID), lambda i, eid, nv: (eid[i], 0, 0)),
            pl.BlockSpec((TM, 1), lambda i, eid, nv: (i, 0)),
        ],
        out_specs=pl.BlockSpec((TM, NCLS), lambda i, eid, nv: (i, 0)),
    )
    return pl.pallas_call(
        _gemm_body,
        grid_spec=grid_spec,
        out_shape=jax.ShapeDtypeStruct((CAP, NCLS), jnp.float32),
    )(tile_eid, nvalid, xs, expert_w1.astype(jnp.bfloat16),
      ln_gamma.reshape(E, 1, HID), ln_beta.reshape(E, 1, HID),
      expert_w2.astype(jnp.bfloat16), gate_s)


_TOK_PER_W = N // NWORK
_CCH = 32


@functools.cache
def _combine_kernel():
    @functools.partial(
        pl.kernel,
        out_type=jax.ShapeDtypeStruct((N, NCLS), jnp.float32),
        mesh=_sc_mesh(),
        scratch_types=[
            pltpu.VMEM((2, _CCH), jnp.int32),
            pltpu.VMEM((2, _CCH, NCLS), jnp.float32),
            pltpu.VMEM((_CCH, NCLS), jnp.float32),
            pltpu.SemaphoreType.DMA,
            pltpu.SemaphoreType.DMA,
        ],
    )
    def _combine(ys_hbm, s0_hbm, s1_hbm, out_hbm, idx_v, a_v, b_v, gsem,
                 wsem):
        wid = lax.axis_index("s") * 2 + lax.axis_index("c")
        base = wid * _TOK_PER_W
        wd = [None, None]
        for c in range(_TOK_PER_W // _CCH):
            b = c & 1
            off = base + c * _CCH
            pltpu.sync_copy(s0_hbm.at[pl.ds(off, _CCH)], idx_v.at[0])
            pltpu.sync_copy(s1_hbm.at[pl.ds(off, _CCH)], idx_v.at[1])
            if wd[b] is not None:
                wd[b].wait()
            g0 = pltpu.async_copy(ys_hbm.at[idx_v.at[0]], a_v.at[b], gsem)
            g1 = pltpu.async_copy(ys_hbm.at[idx_v.at[1]], b_v, gsem)
            g0.wait()
            g1.wait()

            def _add_row(r, _):
                for cc in range(NCLS // 16):
                    sl = pl.ds(cc * 16, 16)
                    a_v[b, r, sl] = a_v[b, r, sl] + b_v[r, sl]
                return 0

            lax.fori_loop(0, _CCH, _add_row, 0)
            wd[b] = pltpu.async_copy(a_v.at[b],
                                     out_hbm.at[pl.ds(off, _CCH)], wsem)
        for d in wd:
            if d is not None:
                d.wait()

    return _combine


def kernel(x, router_w1, router_b1, router_w2, router_b2, router_w3,
           router_b3, expert_w1, ln_gamma, ln_beta, expert_w2):
    i1, i2, g1, g2 = _run_router(x, router_w1, router_b1, router_w2,
                                 router_b2, router_w3, router_b3)

    flat_e = jnp.concatenate([i1[:, 0], i2[:, 0]])
    oh = (flat_e[:, None] == jnp.arange(E, dtype=jnp.int32)[None, :])
    csum = jnp.cumsum(oh.astype(jnp.int32), axis=0)
    rank = jnp.take_along_axis(csum, flat_e[:, None], axis=1)[:, 0] - 1
    counts = csum[-1]
    pc = ((counts + TM - 1) // TM) * TM
    ends = jnp.cumsum(pc)
    poff = ends - pc
    slot = poff[flat_e] + rank
    nvalid_rows = ends[-1]
    nvalid_tiles = (nvalid_rows // TM).astype(jnp.int32)

    tok = jnp.arange(N, dtype=jnp.int32)
    src_row = jnp.zeros((CAP,), jnp.int32).at[slot].set(
        jnp.concatenate([tok, tok]))
    gate_s = jnp.zeros((CAP,), jnp.float32).at[slot].set(
        jnp.concatenate([g1[:, 0], g2[:, 0]]))[:, None]

    tile_start = jnp.arange(NT, dtype=jnp.int32) * TM
    eid_raw = jnp.minimum(
        jnp.searchsorted(ends, tile_start, side="right").astype(jnp.int32),
        E - 1)
    last_eid = jnp.take(eid_raw, jnp.maximum(nvalid_tiles - 1, 0))
    tile_eid = jnp.where(jnp.arange(NT) < nvalid_tiles, eid_raw, last_eid)
    nvalid = nvalid_tiles[None]

    xs = _gather_rows_kernel()(x, src_row)
    ys = _run_experts(xs, expert_w1, ln_gamma, ln_beta, expert_w2, gate_s,
                      tile_eid, nvalid)
    return _combine_kernel()(ys, slot[:N], slot[N:])


# speedup vs baseline: 1.7486x; 1.7486x over previous
import functools

import jax
import jax.numpy as jnp
from jax import lax
from jax.experimental import pallas as pl
from jax.experimental.pallas import tpu as pltpu
from jax.experimental.pallas import tpu_sc as plsc

N = 4096
IN_DIM = 1024
HID = 2048
NCLS = 1024
E = 8
RH = 256

TM = 256
CAP = 2 * N + E * TM
NT = CAP // TM
NWORK = 32
NEG_INF = float("-inf")


@functools.cache
def _sc_mesh():
    return plsc.VectorSubcoreMesh(
        core_axis_name="c", subcore_axis_name="s", num_cores=2,
        num_subcores=16)


_BT = 1024


def _router_body(l_ref, i1_ref, i2_ref, g1_ref, g2_ref):
    logits = l_ref[...]
    ii = lax.broadcasted_iota(jnp.int32, (_BT, E), 1)
    m1 = jnp.max(logits, axis=1, keepdims=True)
    i1 = jnp.min(jnp.where(logits == m1, ii, E), axis=1, keepdims=True)
    l2 = jnp.where(ii == i1, NEG_INF, logits)
    m2 = jnp.max(l2, axis=1, keepdims=True)
    i2 = jnp.min(jnp.where(l2 == m2, ii, E), axis=1, keepdims=True)
    g1 = 1.0 / (1.0 + jnp.exp(m2 - m1))
    i1_ref[...] = i1
    i2_ref[...] = i2
    g1_ref[...] = g1
    g2_ref[...] = 1.0 - g1


def _run_router(x, w1, b1, w2, b2, w3, b3):
    h = jax.nn.relu(x @ w1.T + b1)
    h = jax.nn.relu(h @ w2.T + b2)
    logits = h @ w3.T + b3
    return pl.pallas_call(
        _router_body,
        grid=(N // _BT,),
        in_specs=[pl.BlockSpec((_BT, E), lambda i: (i, 0))],
        out_specs=[pl.BlockSpec((_BT, 1), lambda i: (i, 0))] * 4,
        out_shape=[
            jax.ShapeDtypeStruct((N, 1), jnp.int32),
            jax.ShapeDtypeStruct((N, 1), jnp.int32),
            jax.ShapeDtypeStruct((N, 1), jnp.float32),
            jax.ShapeDtypeStruct((N, 1), jnp.float32),
        ],
    )(logits)


_PPW = 2 * N // NWORK
_DCH = 32


@functools.cache
def _dispatch_kernel():
    @functools.partial(
        pl.kernel,
        out_type=jax.ShapeDtypeStruct((CAP, IN_DIM), jnp.float32),
        mesh=_sc_mesh(),
        scratch_types=[
            pltpu.VMEM((2, _DCH), jnp.int32),
            pltpu.VMEM((2, _DCH, IN_DIM), jnp.float32),
            pltpu.SemaphoreType.DMA,
            pltpu.SemaphoreType.DMA,
        ],
    )
    def _dispatch(x_hbm, slot_hbm, out_hbm, idx_v, rows_v, rsem, ssem):
        wid = lax.axis_index("s") * 2 + lax.axis_index("c")
        nch = _PPW // _DCH
        base_chunk = wid * nch
        pair0 = wid * _PPW

        def tok0(c):
            p = pair0 + c * _DCH
            return lax.rem(p, N)

        pltpu.sync_copy(slot_hbm.at[base_chunk], idx_v.at[0])
        rd = [pltpu.async_copy(x_hbm.at[pl.ds(tok0(0), _DCH)], rows_v.at[0],
                               rsem), None]
        sd = [None, None]
        for c in range(nch):
            b = c & 1
            nb = 1 - b
            rd[b].wait()
            if c + 1 < nch:
                if sd[nb] is not None:
                    sd[nb].wait()
                pltpu.sync_copy(slot_hbm.at[base_chunk + c + 1], idx_v.at[nb])
                rd[nb] = pltpu.async_copy(
                    x_hbm.at[pl.ds(tok0(c + 1), _DCH)], rows_v.at[nb], rsem)
            sd[b] = pltpu.async_copy(rows_v.at[b], out_hbm.at[idx_v.at[b]],
                                     ssem)
        for d in sd:
            if d is not None:
                d.wait()

    return _dispatch


def _gemm_body(eid_ref, nv_ref, xs_ref, w1_ref, g_ref, b_ref, w2_ref,
               out_ref):
    i = pl.program_id(0)

    @pl.when(i < nv_ref[0])
    def _():
        xb = xs_ref[...]
        h = lax.dot_general(xb, w1_ref[0], (((1,), (1,)), ((), ())),
                            preferred_element_type=jnp.float32)
        mu = jnp.mean(h, axis=1, keepdims=True)
        var = jnp.mean((h - mu) * (h - mu), axis=1, keepdims=True)
        h = (h - mu) * lax.rsqrt(var + 1e-5)
        h = h * g_ref[0] + b_ref[0]
        h = jnp.maximum(h, 0.0)
        out_ref[...] = lax.dot_general(
            h, w2_ref[0], (((1,), (1,)), ((), ())),
            preferred_element_type=jnp.float32)


def _run_experts(xs, expert_w1, ln_gamma, ln_beta, expert_w2, tile_eid,
                 nvalid):
    grid_spec = pltpu.PrefetchScalarGridSpec(
        num_scalar_prefetch=2,
        grid=(NT,),
        in_specs=[
            pl.BlockSpec((TM, IN_DIM), lambda i, eid, nv: (i, 0)),
            pl.BlockSpec((1, HID, IN_DIM), lambda i, eid, nv: (eid[i], 0, 0)),
            pl.BlockSpec((1, 1, HID), lambda i, eid, nv: (eid[i], 0, 0)),
            pl.BlockSpec((1, 1, HID), lambda i, eid, nv: (eid[i], 0, 0)),
            pl.BlockSpec((1, NCLS, HID), lambda i, eid, nv: (eid[i], 0, 0)),
        ],
        out_specs=pl.BlockSpec((TM, NCLS), lambda i, eid, nv: (i, 0)),
    )
    return pl.pallas_call(
        _gemm_body,
        grid_spec=grid_spec,
        out_shape=jax.ShapeDtypeStruct((CAP, NCLS), jnp.float32),
    )(tile_eid, nvalid, xs, expert_w1, ln_gamma.reshape(E, 1, HID),
      ln_beta.reshape(E, 1, HID), expert_w2)


_TOK_PER_W = N // NWORK
_CCH = 32


@functools.cache
def _combine_kernel():
    @functools.partial(
        pl.kernel,
        out_type=jax.ShapeDtypeStruct((N, NCLS), jnp.float32),
        mesh=_sc_mesh(),
        scratch_types=[
            pltpu.VMEM((2, _CCH), jnp.int32),
            pltpu.VMEM((2, _CCH, NCLS), jnp.float32),
            pltpu.VMEM((_CCH, NCLS), jnp.float32),
            pltpu.VMEM((_CCH, 16), jnp.float32),
            pltpu.VMEM((_CCH, 16), jnp.float32),
            pltpu.SemaphoreType.DMA,
            pltpu.SemaphoreType.DMA,
        ],
    )
    def _combine(ys_hbm, s0_hbm, s1_hbm, g1_hbm, g2_hbm, out_hbm, idx_v,
                 a_v, b_v, g1_v, g2_v, gsem, wsem):
        wid = lax.axis_index("s") * 2 + lax.axis_index("c")
        base = wid * _TOK_PER_W
        wd = [None, None]
        for c in range(_TOK_PER_W // _CCH):
            b = c & 1
            off = base + c * _CCH
            pltpu.sync_copy(s0_hbm.at[pl.ds(off, _CCH)], idx_v.at[0])
            pltpu.sync_copy(s1_hbm.at[pl.ds(off, _CCH)], idx_v.at[1])
            pltpu.sync_copy(g1_hbm.at[pl.ds(off, _CCH)], g1_v)
            pltpu.sync_copy(g2_hbm.at[pl.ds(off, _CCH)], g2_v)
            if wd[b] is not None:
                wd[b].wait()
            d0 = pltpu.async_copy(ys_hbm.at[idx_v.at[0]], a_v.at[b], gsem)
            d1 = pltpu.async_copy(ys_hbm.at[idx_v.at[1]], b_v, gsem)
            d0.wait()
            d1.wait()

            def _mix_row(r, _):
                ga = g1_v[r, :]
                gb = g2_v[r, :]
                for cc in range(NCLS // 16):
                    sl = pl.ds(cc * 16, 16)
                    a_v[b, r, sl] = ga * a_v[b, r, sl] + gb * b_v[r, sl]
                return 0

            lax.fori_loop(0, _CCH, _mix_row, 0)
            wd[b] = pltpu.async_copy(a_v.at[b],
                                     out_hbm.at[pl.ds(off, _CCH)], wsem)
        for d in wd:
            if d is not None:
                d.wait()

    return _combine


def kernel(x, router_w1, router_b1, router_w2, router_b2, router_w3,
           router_b3, expert_w1, ln_gamma, ln_beta, expert_w2):
    i1, i2, g1, g2 = _run_router(x, router_w1, router_b1, router_w2,
                                 router_b2, router_w3, router_b3)

    flat_e = jnp.concatenate([i1[:, 0], i2[:, 0]])
    oh = (flat_e[:, None] == jnp.arange(E, dtype=jnp.int32)[None, :])
    csum = jnp.cumsum(oh.astype(jnp.int32), axis=0)
    rank = jnp.take_along_axis(csum, flat_e[:, None], axis=1)[:, 0] - 1
    counts = csum[-1]
    pc = ((counts + TM - 1) // TM) * TM
    ends = jnp.cumsum(pc)
    poff = ends - pc
    slot = poff[flat_e] + rank
    nvalid_rows = ends[-1]
    nvalid_tiles = (nvalid_rows // TM).astype(jnp.int32)

    tile_start = jnp.arange(NT, dtype=jnp.int32) * TM
    eid_raw = jnp.minimum(
        jnp.sum((tile_start[:, None] >= ends[None, :]).astype(jnp.int32),
                axis=1), E - 1)
    last_eid = jnp.take(eid_raw, jnp.maximum(nvalid_tiles - 1, 0))
    tile_eid = jnp.where(jnp.arange(NT) < nvalid_tiles, eid_raw, last_eid)
    nvalid = nvalid_tiles[None]

    g1e = jnp.broadcast_to(g1, (N, 16))
    g2e = jnp.broadcast_to(g2, (N, 16))

    xs = _dispatch_kernel()(x, slot.reshape(2 * N // _DCH, _DCH))
    ys = _run_experts(xs, expert_w1, ln_gamma, ln_beta, expert_w2, tile_eid,
                      nvalid)
    return _combine_kernel()(ys, slot[:N], slot[N:], g1e, g2e)
